# baseline (device time: 16991 ns/iter reference)
import jax
import jax.numpy as jnp
from jax import lax
from jax.experimental import pallas as pl
from jax.experimental.pallas import tpu as pltpu

N_DEV = 8
N_CHUNKS = 4


def kernel(x, W, labels):
    t, d = x.shape
    _, v_per = W.shape
    cw = v_per // N_CHUNKS

    def body(x_ref, w_ref, lab_ref, out_ref,
             w_buf, stats_ref, gather_ref, w_sems, send_sems, recv_sems):
        my_pos = lax.axis_index("i")

        barrier_sem = pltpu.get_barrier_semaphore()
        for off in range(1, N_DEV):
            nbr = (my_pos + off) % N_DEV
            pl.semaphore_signal(barrier_sem, inc=1, device_id=(nbr,),
                                device_id_type=pl.DeviceIdType.MESH)

        def w_copy(k):
            return pltpu.make_async_copy(
                w_ref.at[:, pl.ds(k * cw, cw)],
                w_buf.at[k % 2],
                w_sems.at[k % 2],
            )

        w_copy(0).start()

        local_idx = lab_ref[:] - my_pos * v_per
        s_loc = jnp.zeros((t,), jnp.float32)
        c_loc = jnp.zeros((t,), jnp.float32)
        for k in range(N_CHUNKS):
            w_copy(k).wait()
            if k + 1 < N_CHUNKS:
                w_copy(k + 1).start()
            logits = jnp.dot(x_ref[:, :], w_buf[k % 2],
                             preferred_element_type=jnp.float32)
            s_loc = s_loc + jnp.sum(jnp.exp(logits), axis=1)
            cols = k * cw + lax.broadcasted_iota(jnp.int32, (t, cw), 1)
            c_loc = c_loc + jnp.sum(
                jnp.where(cols == local_idx[:, None], logits, 0.0), axis=1)

        stats_ref[:, :] = jnp.concatenate(
            [s_loc[None, :], c_loc[None, :],
             jnp.zeros((6, t), jnp.float32)], axis=0)

        pl.semaphore_wait(barrier_sem, N_DEV - 1)

        rdmas = []
        for off in range(1, N_DEV):
            tgt = (my_pos + off) % N_DEV
            rdma = pltpu.make_async_remote_copy(
                src_ref=stats_ref,
                dst_ref=gather_ref.at[off - 1],
                send_sem=send_sems.at[off - 1],
                recv_sem=recv_sems.at[off - 1],
                device_id=(tgt,),
                device_id_type=pl.DeviceIdType.MESH,
            )
            rdma.start()
            rdmas.append(rdma)
        for rdma in rdmas:
            rdma.wait_recv()

        g = gather_ref[:, :, :]
        s_g = s_loc + jnp.sum(g[:, 0, :], axis=0)
        c_g = c_loc + jnp.sum(g[:, 1, :], axis=0)
        out_ref[:] = jnp.log(s_g) - c_g

        for rdma in rdmas:
            rdma.wait_send()

    return pl.pallas_call(
        body,
        out_shape=jax.ShapeDtypeStruct((t,), jnp.float32),
        in_specs=[
            pl.BlockSpec(memory_space=pltpu.VMEM),
            pl.BlockSpec(memory_space=pl.ANY),
            pl.BlockSpec(memory_space=pltpu.VMEM),
        ],
        out_specs=pl.BlockSpec(memory_space=pltpu.VMEM),
        scratch_shapes=[
            pltpu.VMEM((2, d, cw), jnp.float32),
            pltpu.VMEM((8, t), jnp.float32),
            pltpu.VMEM((N_DEV - 1, 8, t), jnp.float32),
            pltpu.SemaphoreType.DMA((2,)),
            pltpu.SemaphoreType.DMA((N_DEV - 1,)),
            pltpu.SemaphoreType.DMA((N_DEV - 1,)),
        ],
        compiler_params=pltpu.CompilerParams(collective_id=0),
    )(x, W, labels)


# device time: 14001 ns/iter; 1.2136x vs baseline; 1.2136x over previous
import jax
import jax.numpy as jnp
from jax import lax
from jax.experimental import pallas as pl
from jax.experimental.pallas import tpu as pltpu

N_DEV = 8


def kernel(x, W, labels):
    t, d = x.shape
    _, v_per = W.shape

    def body(x_ref, w_ref, lab_ref, out_ref,
             stats_ref, gather_ref, send_sems, recv_sems):
        my_pos = lax.axis_index("i")

        barrier_sem = pltpu.get_barrier_semaphore()
        for off in range(1, N_DEV):
            nbr = (my_pos + off) % N_DEV
            pl.semaphore_signal(barrier_sem, inc=1, device_id=(nbr,),
                                device_id_type=pl.DeviceIdType.MESH)

        logits = jnp.dot(x_ref[:, :], w_ref[:, :],
                         preferred_element_type=jnp.float32)
        s_loc = jnp.sum(jnp.exp(logits), axis=1)

        local_idx = lab_ref[:] - my_pos * v_per
        cols = lax.broadcasted_iota(jnp.int32, (t, v_per), 1)
        c_loc = jnp.sum(jnp.where(cols == local_idx[:, None], logits, 0.0),
                        axis=1)

        stats_ref[:, :] = jnp.concatenate(
            [s_loc[None, :], c_loc[None, :],
             jnp.zeros((6, t), jnp.float32)], axis=0)

        pl.semaphore_wait(barrier_sem, N_DEV - 1)

        rdmas = []
        for off in range(1, N_DEV):
            tgt = (my_pos + off) % N_DEV
            rdma = pltpu.make_async_remote_copy(
                src_ref=stats_ref,
                dst_ref=gather_ref.at[off - 1],
                send_sem=send_sems.at[off - 1],
                recv_sem=recv_sems.at[off - 1],
                device_id=(tgt,),
                device_id_type=pl.DeviceIdType.MESH,
            )
            rdma.start()
            rdmas.append(rdma)
        for rdma in rdmas:
            rdma.wait_recv()

        g = gather_ref[:, :, :]
        s_g = s_loc + jnp.sum(g[:, 0, :], axis=0)
        c_g = c_loc + jnp.sum(g[:, 1, :], axis=0)
        out_ref[:] = jnp.log(s_g) - c_g

        for rdma in rdmas:
            rdma.wait_send()

    return pl.pallas_call(
        body,
        out_shape=jax.ShapeDtypeStruct((t,), jnp.float32),
        in_specs=[
            pl.BlockSpec(memory_space=pltpu.VMEM),
            pl.BlockSpec(memory_space=pltpu.VMEM),
            pl.BlockSpec(memory_space=pltpu.VMEM),
        ],
        out_specs=pl.BlockSpec(memory_space=pltpu.VMEM),
        scratch_shapes=[
            pltpu.VMEM((8, t), jnp.float32),
            pltpu.VMEM((N_DEV - 1, 8, t), jnp.float32),
            pltpu.SemaphoreType.DMA((N_DEV - 1,)),
            pltpu.SemaphoreType.DMA((N_DEV - 1,)),
        ],
        compiler_params=pltpu.CompilerParams(collective_id=0),
    )(x, W, labels)


# device time: 13839 ns/iter; 1.2278x vs baseline; 1.0117x over previous
import jax
import jax.numpy as jnp
from jax import lax
from jax.experimental import pallas as pl
from jax.experimental.pallas import tpu as pltpu

N_DEV = 8


def kernel(x, W, labels):
    t, d = x.shape
    _, v_per = W.shape

    def body(x_ref, w_ref, lab_ref, out_ref,
             stats_ref, gather_ref, send_sems, recv_sems):
        my_pos = lax.axis_index("i")

        barrier_sem = pltpu.get_barrier_semaphore()
        for off in range(1, N_DEV):
            nbr = (my_pos + off) % N_DEV
            pl.semaphore_signal(barrier_sem, inc=1, device_id=(nbr,),
                                device_id_type=pl.DeviceIdType.MESH)

        logits = jnp.dot(x_ref[:, :], w_ref[:, :],
                         preferred_element_type=jnp.float32)
        s_loc = jnp.sum(jnp.exp(logits), axis=1)

        local_idx = lab_ref[:] - my_pos * v_per
        cols = lax.broadcasted_iota(jnp.int32, (t, v_per), 1)
        c_loc = jnp.sum(jnp.where(cols == local_idx[:, None], logits, 0.0),
                        axis=1)

        stats_ref[:, :] = jnp.concatenate(
            [s_loc[None, :], c_loc[None, :]], axis=0)

        pl.semaphore_wait(barrier_sem, N_DEV - 1)

        rdmas = []
        for off in range(1, N_DEV):
            tgt = (my_pos + off) % N_DEV
            rdma = pltpu.make_async_remote_copy(
                src_ref=stats_ref,
                dst_ref=gather_ref.at[off - 1],
                send_sem=send_sems.at[off - 1],
                recv_sem=recv_sems.at[off - 1],
                device_id=(tgt,),
                device_id_type=pl.DeviceIdType.MESH,
            )
            rdma.start()
            rdmas.append(rdma)
        for rdma in rdmas:
            rdma.wait_recv()

        g = gather_ref[:, :, :]
        s_g = s_loc + jnp.sum(g[:, 0, :], axis=0)
        c_g = c_loc + jnp.sum(g[:, 1, :], axis=0)
        out_ref[:] = jnp.log(s_g) - c_g

        for rdma in rdmas:
            rdma.wait_send()

    return pl.pallas_call(
        body,
        out_shape=jax.ShapeDtypeStruct((t,), jnp.float32),
        in_specs=[
            pl.BlockSpec(memory_space=pltpu.VMEM),
            pl.BlockSpec(memory_space=pltpu.VMEM),
            pl.BlockSpec(memory_space=pltpu.VMEM),
        ],
        out_specs=pl.BlockSpec(memory_space=pltpu.VMEM),
        scratch_shapes=[
            pltpu.VMEM((2, t), jnp.float32),
            pltpu.VMEM((N_DEV - 1, 2, t), jnp.float32),
            pltpu.SemaphoreType.DMA((N_DEV - 1,)),
            pltpu.SemaphoreType.DMA((N_DEV - 1,)),
        ],
        compiler_params=pltpu.CompilerParams(collective_id=0),
    )(x, W, labels)
